# baseline (speedup 1.0000x reference)
import functools

import jax
import jax.numpy as jnp
from jax.experimental import pallas as pl
from jax.experimental.pallas import tpu as pltpu

N = 4096
NFEAT = 512
NHID = 256
NCLASS = 64

BM1 = 1024
NB1 = N // BM1
BMZ = 1024
NBZ = N // BMZ
BMD = 2048
NBD = N // BMD

ADJ_SCALE = 2048.0
F8 = jnp.float8_e4m3fn

_DN = (((1,), (1,)), ((), ()))


def _pass1(adj_ref, x_ref, w1_ref, b1_ref, w2_ref, s2_ref, adj8_ref):
    a32 = adj_ref[...]
    a = a32.astype(jnp.bfloat16)
    adj8_ref[...] = (a32 * ADJ_SCALE).astype(F8)
    xb = x_ref[...].astype(jnp.bfloat16)
    t = jnp.dot(a, xb, preferred_element_type=jnp.float32).astype(jnp.bfloat16)
    w1 = w1_ref[...].astype(jnp.bfloat16)
    h = jnp.dot(t, w1, preferred_element_type=jnp.float32)
    h = jnp.maximum(h + b1_ref[...], 0.0).astype(jnp.bfloat16)
    w2 = w2_ref[...].astype(jnp.bfloat16)
    s2_ref[...] = jnp.dot(h, w2, preferred_element_type=jnp.float32).astype(
        jnp.bfloat16)


def _pass2(adj8_ref, s2_ref, b2_ref, out_ref, z_ref):
    k = pl.program_id(0)

    @pl.when(k < NBZ)
    def _z_phase():
        a = adj8_ref[...].astype(jnp.bfloat16)
        acc = jnp.dot(a, s2_ref[...], preferred_element_type=jnp.float32)
        zb = acc * (1.0 / ADJ_SCALE) + b2_ref[...]
        z_ref[pl.ds(k * BMZ, BMZ), :] = zb.astype(jnp.bfloat16)

    @pl.when(k >= NBZ)
    def _decode_phase():
        d = k - NBZ
        i = d // NBD
        j = d % NBD
        zi = z_ref[pl.ds(i * BMD, BMD), :]
        zj = z_ref[pl.ds(j * BMD, BMD), :]
        t = jax.lax.dot_general(zi, zj, _DN, preferred_element_type=jnp.float32)
        out_ref[...] = 0.5 * (1.0 + jnp.tanh(0.5 * t))


def _p2_adj8_map(k):
    return (jnp.minimum(k, NBZ - 1), 0)


def _p2_out_map(k):
    d = jnp.maximum(k - NBZ, 0)
    return (d // NBD, d % NBD)


@functools.partial(jax.jit)
def kernel(x, adj, W1, b1, W2, b2):
    b1r = b1.reshape(1, NHID)
    b2r = b2.reshape(1, NCLASS)

    s2, adj8 = pl.pallas_call(
        _pass1,
        grid=(NB1,),
        in_specs=[
            pl.BlockSpec((BM1, N), lambda i: (i, 0)),
            pl.BlockSpec((N, NFEAT), lambda i: (0, 0)),
            pl.BlockSpec((NFEAT, NHID), lambda i: (0, 0)),
            pl.BlockSpec((1, NHID), lambda i: (0, 0)),
            pl.BlockSpec((NHID, NCLASS), lambda i: (0, 0)),
        ],
        out_specs=[
            pl.BlockSpec((BM1, NCLASS), lambda i: (i, 0)),
            pl.BlockSpec((BM1, N), lambda i: (i, 0)),
        ],
        out_shape=[
            jax.ShapeDtypeStruct((N, NCLASS), jnp.bfloat16),
            jax.ShapeDtypeStruct((N, N), F8),
        ],
    )(adj, x, W1, b1r, W2)

    a_pred = pl.pallas_call(
        _pass2,
        grid=(NBZ + NBD * NBD,),
        in_specs=[
            pl.BlockSpec((BMZ, N), _p2_adj8_map),
            pl.BlockSpec((N, NCLASS), lambda k: (0, 0)),
            pl.BlockSpec((1, NCLASS), lambda k: (0, 0)),
        ],
        out_specs=pl.BlockSpec((BMD, BMD), _p2_out_map),
        out_shape=jax.ShapeDtypeStruct((N, N), jnp.float32),
        scratch_shapes=[pltpu.VMEM((N, NCLASS), jnp.bfloat16)],
    )(adj8, s2, b2r)

    return a_pred


# speedup vs baseline: 1.0147x; 1.0147x over previous
import functools

import jax
import jax.numpy as jnp
from jax.experimental import pallas as pl
from jax.experimental.pallas import tpu as pltpu

N = 4096
NFEAT = 512
NHID = 256
NCLASS = 64

BM1 = 1024
NB1 = N // BM1
BMZ = 1024
NBZ = N // BMZ
BMD = 2048
NBD = N // BMD

ADJ_SCALE = 2048.0
F8 = jnp.float8_e4m3fn

_DN = (((1,), (1,)), ((), ()))


def _pass1(adj_ref, x_ref, w1_ref, b1_ref, w2_ref, s2_ref, adj8_ref, s1_ref):
    @pl.when(pl.program_id(0) == 0)
    def _compute_s1():
        xb = x_ref[...].astype(jnp.bfloat16)
        w1 = w1_ref[...].astype(jnp.bfloat16)
        s1_ref[...] = jnp.dot(xb, w1, preferred_element_type=jnp.float32
                              ).astype(jnp.bfloat16)

    a32 = adj_ref[...]
    a = a32.astype(jnp.bfloat16)
    adj8_ref[...] = (a32 * ADJ_SCALE).astype(F8)
    h = jnp.dot(a, s1_ref[...], preferred_element_type=jnp.float32)
    h = jnp.maximum(h + b1_ref[...], 0.0).astype(jnp.bfloat16)
    w2 = w2_ref[...].astype(jnp.bfloat16)
    s2_ref[...] = jnp.dot(h, w2, preferred_element_type=jnp.float32).astype(
        jnp.bfloat16)


def _pass2(adj8_ref, s2_ref, b2_ref, out_ref, z_ref):
    k = pl.program_id(0)

    @pl.when(k < NBZ)
    def _z_phase():
        a = adj8_ref[...].astype(jnp.bfloat16)
        acc = jnp.dot(a, s2_ref[...], preferred_element_type=jnp.float32)
        zb = acc * (1.0 / ADJ_SCALE) + b2_ref[...]
        z_ref[pl.ds(k * BMZ, BMZ), :] = zb.astype(jnp.bfloat16)

    @pl.when(k >= NBZ)
    def _decode_phase():
        d = k - NBZ
        i = d // NBD
        j = d % NBD
        zi = z_ref[pl.ds(i * BMD, BMD), :]
        zj = z_ref[pl.ds(j * BMD, BMD), :]
        t = jax.lax.dot_general(zi, zj, _DN, preferred_element_type=jnp.float32)
        out_ref[...] = 0.5 * (1.0 + jnp.tanh(0.5 * t))


def _p2_adj8_map(k):
    return (jnp.minimum(k, NBZ - 1), 0)


def _p2_out_map(k):
    d = jnp.maximum(k - NBZ, 0)
    return (d // NBD, d % NBD)


@functools.partial(jax.jit)
def kernel(x, adj, W1, b1, W2, b2):
    b1r = b1.reshape(1, NHID)
    b2r = b2.reshape(1, NCLASS)

    s2, adj8 = pl.pallas_call(
        _pass1,
        grid=(NB1,),
        in_specs=[
            pl.BlockSpec((BM1, N), lambda i: (i, 0)),
            pl.BlockSpec((N, NFEAT), lambda i: (0, 0)),
            pl.BlockSpec((NFEAT, NHID), lambda i: (0, 0)),
            pl.BlockSpec((1, NHID), lambda i: (0, 0)),
            pl.BlockSpec((NHID, NCLASS), lambda i: (0, 0)),
        ],
        out_specs=[
            pl.BlockSpec((BM1, NCLASS), lambda i: (i, 0)),
            pl.BlockSpec((BM1, N), lambda i: (i, 0)),
        ],
        out_shape=[
            jax.ShapeDtypeStruct((N, NCLASS), jnp.bfloat16),
            jax.ShapeDtypeStruct((N, N), F8),
        ],
        scratch_shapes=[pltpu.VMEM((N, NHID), jnp.bfloat16)],
    )(adj, x, W1, b1r, W2)

    a_pred = pl.pallas_call(
        _pass2,
        grid=(NBZ + NBD * NBD,),
        in_specs=[
            pl.BlockSpec((BMZ, N), _p2_adj8_map),
            pl.BlockSpec((N, NCLASS), lambda k: (0, 0)),
            pl.BlockSpec((1, NCLASS), lambda k: (0, 0)),
        ],
        out_specs=pl.BlockSpec((BMD, BMD), _p2_out_map),
        out_shape=jax.ShapeDtypeStruct((N, N), jnp.float32),
        scratch_shapes=[pltpu.VMEM((N, NCLASS), jnp.bfloat16)],
    )(adj8, s2, b2r)

    return a_pred


# speedup vs baseline: 1.1602x; 1.1434x over previous
import functools

import jax
import jax.numpy as jnp
from jax.experimental import pallas as pl
from jax.experimental.pallas import tpu as pltpu

N = 4096
NFEAT = 512
NHID = 256
NCLASS = 64

BM1 = 512
NB1 = N // BM1
BMD = 1024
NBD = N // BMD

ADJ_SCALE = 2048.0
F8 = jnp.float8_e4m3fn

_DN = (((1,), (1,)), ((), ()))


def _body(adj_ref, x_ref, w1_ref, b1_ref, w2_ref, b2_ref, out_ref,
          adj8_v, s1_v, s2_v, z_v):
    k = pl.program_id(0)

    @pl.when(k == 0)
    def _compute_s1():
        xb = x_ref[...].astype(jnp.bfloat16)
        w1 = w1_ref[...].astype(jnp.bfloat16)
        s1_v[...] = jnp.dot(xb, w1, preferred_element_type=jnp.float32
                            ).astype(jnp.bfloat16)

    @pl.when(k < NB1)
    def _layer_phase():
        a32 = adj_ref[...]
        adj8_v[pl.ds(k * BM1, BM1), :] = (a32 * ADJ_SCALE).astype(F8)
        a = a32.astype(jnp.bfloat16)
        h = jnp.dot(a, s1_v[...], preferred_element_type=jnp.float32)
        h = jnp.maximum(h + b1_ref[...], 0.0).astype(jnp.bfloat16)
        w2 = w2_ref[...].astype(jnp.bfloat16)
        s2_v[pl.ds(k * BM1, BM1), :] = jnp.dot(
            h, w2, preferred_element_type=jnp.float32).astype(jnp.bfloat16)

    @pl.when(k >= NB1)
    def _decode_phase():
        d = k - NB1

        @pl.when(d < NBD)
        def _z_diag():
            a = adj8_v[pl.ds(d * BMD, BMD), :].astype(jnp.bfloat16)
            acc = jnp.dot(a, s2_v[...], preferred_element_type=jnp.float32)
            zb = acc * (1.0 / ADJ_SCALE) + b2_ref[...]
            z_v[pl.ds(d * BMD, BMD), :] = zb.astype(jnp.bfloat16)

        i, j = _tile_ij(d)
        zi = z_v[pl.ds(i * BMD, BMD), :]
        zj = z_v[pl.ds(j * BMD, BMD), :]
        t = jax.lax.dot_general(zi, zj, _DN, preferred_element_type=jnp.float32)
        out_ref[...] = 0.5 * (1.0 + jnp.tanh(0.5 * t))


def _tile_ij(d):
    e = jnp.maximum(d - NBD, 0)
    i_off = e // (NBD - 1)
    jj = e % (NBD - 1)
    j_off = jj + (jj >= i_off).astype(jj.dtype)
    on_diag = d < NBD
    i = jnp.where(on_diag, d, i_off)
    j = jnp.where(on_diag, d, j_off)
    return i, j


def _out_map(k):
    d = jnp.maximum(k - NB1, 0)
    return _tile_ij(d)


@functools.partial(jax.jit)
def kernel(x, adj, W1, b1, W2, b2):
    b1r = b1.reshape(1, NHID)
    b2r = b2.reshape(1, NCLASS)

    a_pred = pl.pallas_call(
        _body,
        grid=(NB1 + NBD * NBD,),
        in_specs=[
            pl.BlockSpec((BM1, N), lambda k: (jnp.minimum(k, NB1 - 1), 0)),
            pl.BlockSpec((N, NFEAT), lambda k: (0, 0)),
            pl.BlockSpec((NFEAT, NHID), lambda k: (0, 0)),
            pl.BlockSpec((1, NHID), lambda k: (0, 0)),
            pl.BlockSpec((NHID, NCLASS), lambda k: (0, 0)),
            pl.BlockSpec((1, NCLASS), lambda k: (0, 0)),
        ],
        out_specs=pl.BlockSpec((BMD, BMD), _out_map),
        out_shape=jax.ShapeDtypeStruct((N, N), jnp.float32),
        scratch_shapes=[
            pltpu.VMEM((N, N), F8),
            pltpu.VMEM((N, NHID), jnp.bfloat16),
            pltpu.VMEM((N, NCLASS), jnp.bfloat16),
            pltpu.VMEM((N, NCLASS), jnp.bfloat16),
        ],
    )(adj, x, W1, b1r, W2, b2r)

    return a_pred
